# baseline (speedup 1.0000x reference)
import functools

import jax
import jax.numpy as jnp
from jax import lax
from jax.experimental import pallas as pl
from jax.experimental.pallas import tpu as pltpu
from jax.experimental.pallas import tpu_sc as plsc

_NUM_ROWS = 32
_NUM_COLS = 32
_EMBED_DIM = 768
_LANES = 16
_CHUNKS = _EMBED_DIM // _LANES
_GROUP = 8
_NGROUPS = _NUM_COLS // _GROUP

_mesh = plsc.VectorSubcoreMesh(core_axis_name="c", subcore_axis_name="s")


@functools.partial(
    pl.kernel,
    mesh=_mesh,
    out_type=jax.ShapeDtypeStruct((_NUM_ROWS * _NUM_COLS, _EMBED_DIM), jnp.float32),
    scratch_types=[
        pltpu.VMEM((8, _EMBED_DIM), jnp.float32),
        pltpu.VMEM((1 + _NUM_COLS, _EMBED_DIM), jnp.float32),
        pltpu.VMEM((_NUM_COLS, _EMBED_DIM), jnp.float32),
        pltpu.SemaphoreType.DMA,
        pltpu.SemaphoreType.DMA,
        pltpu.SemaphoreType.DMA,
    ],
)
def _pos2d(wrow_hbm, wcol_hbm, out_hbm, wr8_v, wc_v, out_v, rsem, csem, osem):
    num_cores = 2
    wid = lax.axis_index("s") * num_cores + lax.axis_index("c")
    base8 = ((wid + 1) // 8) * 8
    local = wid + 1 - base8
    row_cp = pltpu.async_copy(wrow_hbm.at[pl.ds(base8, 8)], wr8_v, rsem)
    col_cp = pltpu.async_copy(wcol_hbm, wc_v, csem)
    row_cp.wait()
    col_cp.wait()

    wr_regs = [wr8_v[local, pl.ds(j * _LANES, _LANES)] for j in range(_CHUNKS)]

    for g in range(_NGROUPS):
        def col_body(c, _):
            for j in range(_CHUNKS):
                sl = pl.ds(j * _LANES, _LANES)
                out_v[c, sl] = wc_v[c + 1, sl] + wr_regs[j]
            return 0

        lax.fori_loop(g * _GROUP, (g + 1) * _GROUP, col_body, 0)
        pltpu.async_copy(
            out_v.at[pl.ds(g * _GROUP, _GROUP)],
            out_hbm.at[pl.ds(wid * _NUM_COLS + g * _GROUP, _GROUP)],
            osem,
        )
    pltpu.make_async_copy(
        out_v, out_hbm.at[pl.ds(wid * _NUM_COLS, _NUM_COLS)], osem
    ).wait()


def kernel(input, W_row, W_col):
    del input
    out = _pos2d(W_row, W_col)
    return out.reshape(1, _NUM_ROWS * _NUM_COLS, _EMBED_DIM)


# speedup vs baseline: 1.3602x; 1.3602x over previous
import functools

import jax
import jax.numpy as jnp
from jax import lax
from jax.experimental import pallas as pl
from jax.experimental.pallas import tpu as pltpu
from jax.experimental.pallas import tpu_sc as plsc

_NUM_ROWS = 32
_NUM_COLS = 32
_EMBED_DIM = 768
_LANES = 16
_CHUNKS = _EMBED_DIM // _LANES
_RPT = 2

_mesh = plsc.VectorSubcoreMesh(
    core_axis_name="c", subcore_axis_name="s", num_cores=1
)


@functools.partial(
    pl.kernel,
    mesh=_mesh,
    out_type=jax.ShapeDtypeStruct((_NUM_ROWS * _NUM_COLS, _EMBED_DIM), jnp.float32),
    scratch_types=[
        pltpu.VMEM((_RPT * _EMBED_DIM,), jnp.float32),
        pltpu.VMEM((_RPT * _NUM_COLS, _EMBED_DIM), jnp.float32),
        pltpu.SemaphoreType.DMA,
        pltpu.SemaphoreType.DMA,
    ],
)
def _pos2d(wrow_hbm, wcol_hbm, out_hbm, wr_v, out_v, rsem, csem):
    wid = lax.axis_index("s")
    row_cp = pltpu.async_copy(
        wrow_hbm.at[pl.ds(wid * (_RPT * _EMBED_DIM), _RPT * _EMBED_DIM)], wr_v, rsem
    )
    init0 = pltpu.async_copy(wcol_hbm, out_v.at[pl.ds(0, _NUM_COLS)], csem)
    init1 = pltpu.async_copy(wcol_hbm, out_v.at[pl.ds(_NUM_COLS, _NUM_COLS)], csem)
    row_cp.wait()
    init0.wait()
    init1.wait()

    for r in range(_RPT):
        wr_regs = [
            wr_v[pl.ds(r * _EMBED_DIM + j * _LANES, _LANES)] for j in range(_CHUNKS)
        ]

        def col_body(c, _):
            for j in range(_CHUNKS):
                plsc.addupdate(out_v.at[c, pl.ds(j * _LANES, _LANES)], wr_regs[j])
            return 0

        lax.fori_loop(r * _NUM_COLS, (r + 1) * _NUM_COLS, col_body, 0)

    pltpu.sync_copy(out_v, out_hbm.at[pl.ds(wid * (_RPT * _NUM_COLS), _RPT * _NUM_COLS)])


def kernel(input, W_row, W_col):
    del input
    wr = W_row[1 : 1 + _NUM_ROWS].reshape(_NUM_ROWS * _EMBED_DIM)
    wc = W_col[1 : 1 + _NUM_COLS]
    out = _pos2d(wr, wc)
    return out.reshape(1, _NUM_ROWS * _NUM_COLS, _EMBED_DIM)


# speedup vs baseline: 1.4083x; 1.0353x over previous
import functools

import jax
import jax.numpy as jnp
from jax import lax
from jax.experimental import pallas as pl
from jax.experimental.pallas import tpu as pltpu
from jax.experimental.pallas import tpu_sc as plsc

_NUM_ROWS = 32
_NUM_COLS = 32
_EMBED_DIM = 768
_LANES = 16
_CHUNKS = _EMBED_DIM // _LANES
_RPT = 2

_mesh = plsc.VectorSubcoreMesh(
    core_axis_name="c", subcore_axis_name="s", num_cores=1
)


@functools.partial(
    pl.kernel,
    mesh=_mesh,
    out_type=jax.ShapeDtypeStruct((_NUM_ROWS * _NUM_COLS, _EMBED_DIM), jnp.float32),
    scratch_types=[
        pltpu.VMEM((_RPT * _EMBED_DIM,), jnp.float32),
        pltpu.VMEM((_RPT * _NUM_COLS, _EMBED_DIM), jnp.float32),
        pltpu.SemaphoreType.DMA,
        pltpu.SemaphoreType.DMA,
        pltpu.SemaphoreType.DMA,
    ],
)
def _pos2d(wrow_hbm, wcol_hbm, out_hbm, wr_v, out_v, rsem, csem, osem):
    wid = lax.axis_index("s")
    row_cp = pltpu.async_copy(
        wrow_hbm.at[pl.ds(wid * (_RPT * _EMBED_DIM), _RPT * _EMBED_DIM)], wr_v, rsem
    )
    init0 = pltpu.async_copy(wcol_hbm, out_v.at[pl.ds(0, _NUM_COLS)], csem)
    init1 = pltpu.async_copy(wcol_hbm, out_v.at[pl.ds(_NUM_COLS, _NUM_COLS)], csem)
    row_cp.wait()
    init0.wait()
    init1.wait()

    for r in range(_RPT):
        wr_regs = [
            wr_v[pl.ds(r * _EMBED_DIM + j * _LANES, _LANES)] for j in range(_CHUNKS)
        ]

        def col_body(c, _):
            for j in range(_CHUNKS):
                plsc.addupdate(out_v.at[c, pl.ds(j * _LANES, _LANES)], wr_regs[j])
            return 0

        lax.fori_loop(r * _NUM_COLS, (r + 1) * _NUM_COLS, col_body, 0)
        pltpu.async_copy(
            out_v.at[pl.ds(r * _NUM_COLS, _NUM_COLS)],
            out_hbm.at[pl.ds(wid * (_RPT * _NUM_COLS) + r * _NUM_COLS, _NUM_COLS)],
            osem,
        )

    pltpu.make_async_copy(
        out_v, out_hbm.at[pl.ds(wid * (_RPT * _NUM_COLS), _RPT * _NUM_COLS)], osem
    ).wait()


def kernel(input, W_row, W_col):
    del input
    wr = W_row[1 : 1 + _NUM_ROWS].reshape(_NUM_ROWS * _EMBED_DIM)
    wc = W_col[1 : 1 + _NUM_COLS]
    out = _pos2d(wr, wc)
    return out.reshape(1, _NUM_ROWS * _NUM_COLS, _EMBED_DIM)
